# baseline (speedup 1.0000x reference)
import functools

import jax
import jax.numpy as jnp
from jax import lax
from jax.experimental import pallas as pl
from jax.experimental.pallas import tpu as pltpu
from jax.experimental.pallas import tpu_sc as plsc

N = 10000
E = 320000
D_IN = 128
D_H = 128
D_H2 = 64
DDEG = 8

NC = 2
NS = 16
NW = NC * NS
CBP = 128
CHP = 80
EPW = CHP * CBP
EPAD = NW * EPW - E
NPAD = 10240
RPT = NPAD // NS

_SC_MESH = dict(core_axis_name="c", subcore_axis_name="s",
                num_cores=NC, num_subcores=NS)
_SC_PARAMS = pltpu.CompilerParams(use_tc_tiling_on_sc=False)


def _make_agg(D):
    @functools.partial(
        pl.kernel,
        out_type=jax.ShapeDtypeStruct((NC, NPAD, D), jnp.float32),
        mesh=plsc.VectorSubcoreMesh(**_SC_MESH),
        scratch_types=[
            pltpu.VMEM((CBP,), jnp.int32),
            pltpu.VMEM((CBP,), jnp.int32),
            pltpu.VMEM((CBP, D), jnp.float32),
            pltpu.VMEM_SHARED((NPAD, D), jnp.float32),
            pltpu.SemaphoreType.DMA,
            pltpu.SemaphoreType.DMA,
        ],
        compiler_params=_SC_PARAMS,
    )
    def agg(y_hbm, src_hbm, dst_hbm, zero_hbm, out_hbm, sidx, didx,
            rows_v, acc_sh, gsem, ssem):
        c = lax.axis_index("c")
        s = lax.axis_index("s")
        w = c * NS + s
        r0 = s * RPT
        pltpu.sync_copy(zero_hbm, acc_sh.at[pl.ds(r0, RPT)])
        plsc.subcore_barrier()

        def body(k, carry):
            pltpu.sync_copy(src_hbm.at[w, k], sidx)
            pltpu.sync_copy(dst_hbm.at[w, k], didx)
            pltpu.async_copy(y_hbm.at[sidx], rows_v, gsem)
            pltpu.make_async_copy(y_hbm.at[sidx], rows_v, gsem).wait()
            pltpu.async_copy(rows_v, acc_sh.at[didx], ssem, add=True)
            pltpu.make_async_copy(rows_v, acc_sh.at[didx], ssem).wait()
            return carry

        lax.fori_loop(0, CHP, body, 0)
        plsc.subcore_barrier()
        pltpu.sync_copy(acc_sh.at[pl.ds(r0, RPT)],
                        out_hbm.at[c].at[pl.ds(r0, RPT)])

    return agg


_agg128 = _make_agg(128)
_agg64 = _make_agg(64)
_agg8 = _make_agg(8)


@functools.partial(
    pl.kernel,
    out_type=jax.ShapeDtypeStruct((NC, NPAD, DDEG), jnp.float32),
    mesh=plsc.VectorSubcoreMesh(**_SC_MESH),
    scratch_types=[
        pltpu.VMEM((CHP, CBP), jnp.int32),
        pltpu.VMEM((CBP, DDEG), jnp.float32),
        pltpu.VMEM_SHARED((NPAD, DDEG), jnp.float32),
        pltpu.SemaphoreType.DMA,
    ],
    compiler_params=_SC_PARAMS,
)
def _deg_sc(ones_hbm, dst_hbm, zero_hbm, out_hbm, dst_st, rows_v, acc_sh, sem):
    c = lax.axis_index("c")
    s = lax.axis_index("s")
    w = c * NS + s
    r0 = s * RPT
    pltpu.sync_copy(dst_hbm.at[w], dst_st)
    pltpu.sync_copy(zero_hbm, acc_sh.at[pl.ds(r0, RPT)])
    pltpu.sync_copy(ones_hbm, rows_v)
    plsc.subcore_barrier()

    def body(j, carry):
        pltpu.async_copy(rows_v, acc_sh.at[dst_st.at[j]], sem, add=True)
        return carry

    lax.fori_loop(0, CHP, body, 0)

    def drain(j, carry):
        pltpu.make_async_copy(rows_v, acc_sh.at[dst_st.at[0]], sem).wait()
        return carry

    lax.fori_loop(0, CHP, drain, 0)
    plsc.subcore_barrier()
    pltpu.sync_copy(acc_sh.at[pl.ds(r0, RPT)],
                    out_hbm.at[c].at[pl.ds(r0, RPT)])


_RB = 1000
_GRID = N // _RB


def _part_spec(core, d):
    return pl.BlockSpec((1, _RB, d), lambda i, c=core: (c, i, 0))


def _row_spec(d):
    return pl.BlockSpec((_RB, d), lambda i: (i, 0))


def _full_spec(shape):
    return pl.BlockSpec(shape, lambda i: tuple(0 for _ in shape))


def _bf16_dot(a, b):
    return jnp.dot(a.astype(jnp.bfloat16), b.astype(jnp.bfloat16),
                   preferred_element_type=jnp.float32)


def _k0_body(x, w1, xw_ref):
    xw_ref[...] = _bf16_dot(x[...], w1[...])


def _k1_body(d0, d1, xw, y_ref, dis_ref):
    deg = d0[0][:, 0:1] + d1[0][:, 0:1] + 1.0
    dis = 1.0 / jnp.sqrt(deg)
    dis_ref[...] = dis
    y_ref[...] = dis * xw[...]


def _k2_body(p0, p1, y1, dis, b1, w2, y2_ref):
    h = jnp.maximum(dis[...] * (p0[0] + p1[0] + y1[...]) + b1[...], 0.0)
    y2_ref[...] = dis[...] * _bf16_dot(h, w2[...])


def _k3_body(q0, q1, y2, dis, b2, w3p, y3_ref):
    h = jnp.maximum(dis[...] * (q0[0] + q1[0] + y2[...]) + b2[...], 0.0)
    y3_ref[...] = dis[...] * _bf16_dot(h, w3p[...])


def _k4_body(r0, r1, y3, dis, b3, out_ref):
    t = r0[0][:, 0:1] + r1[0][:, 0:1] + y3[:, 0:1]
    out_ref[...] = dis[...] * t + b3[...]


def kernel(x, edge_index, W1, b1, W2, b2, W3, b3):
    f32 = jnp.float32
    i32 = jnp.int32
    src = jnp.concatenate(
        [edge_index[0], jnp.zeros((EPAD,), i32)]).reshape(NW, CHP, CBP)
    dst = jnp.concatenate(
        [edge_index[1], jnp.full((EPAD,), N, i32)]).reshape(NW, CHP, CBP)
    onesb = jnp.ones((CBP, DDEG), f32)
    zdeg = jnp.zeros((RPT, DDEG), f32)
    z64 = jnp.zeros((RPT, 64), f32)
    z128 = jnp.zeros((RPT, 128), f32)
    b1r = b1.reshape(1, D_H)
    b2r = b2.reshape(1, D_H2)
    b3r = b3.reshape(1, 1)

    degp = _deg_sc(onesb, dst, zdeg)
    xw1 = pl.pallas_call(
        _k0_body,
        grid=(_GRID,),
        in_specs=[_row_spec(D_IN), _full_spec((D_IN, D_H))],
        out_specs=_row_spec(D_H),
        out_shape=jax.ShapeDtypeStruct((N, D_H), f32),
    )(x, W1)
    y1, dis = pl.pallas_call(
        _k1_body,
        grid=(_GRID,),
        in_specs=[_part_spec(0, DDEG), _part_spec(1, DDEG), _row_spec(D_H)],
        out_specs=[_row_spec(D_H), _row_spec(1)],
        out_shape=[jax.ShapeDtypeStruct((N, D_H), f32),
                   jax.ShapeDtypeStruct((N, 1), f32)],
    )(degp, degp, xw1)

    p = _agg128(y1, src, dst, z128)
    y2 = pl.pallas_call(
        _k2_body,
        grid=(_GRID,),
        in_specs=[_part_spec(0, D_H), _part_spec(1, D_H), _row_spec(D_H),
                  _row_spec(1), _full_spec((1, D_H)), _full_spec((D_H, D_H2))],
        out_specs=_row_spec(D_H2),
        out_shape=jax.ShapeDtypeStruct((N, D_H2), f32),
    )(p, p, y1, dis, b1r, W2)

    q = _agg64(y2, src, dst, z64)
    w3p = jnp.pad(W3, ((0, 0), (0, DDEG - 1)))
    y3 = pl.pallas_call(
        _k3_body,
        grid=(_GRID,),
        in_specs=[_part_spec(0, D_H2), _part_spec(1, D_H2), _row_spec(D_H2),
                  _row_spec(1), _full_spec((1, D_H2)),
                  _full_spec((D_H2, DDEG))],
        out_specs=_row_spec(DDEG),
        out_shape=jax.ShapeDtypeStruct((N, DDEG), f32),
    )(q, q, y2, dis, b2r, w3p)

    r = _agg8(y3, src, dst, zdeg)
    out = pl.pallas_call(
        _k4_body,
        grid=(_GRID,),
        in_specs=[_part_spec(0, DDEG), _part_spec(1, DDEG), _row_spec(DDEG),
                  _row_spec(1), _full_spec((1, 1))],
        out_specs=_row_spec(1),
        out_shape=jax.ShapeDtypeStruct((N, 1), f32),
    )(r, r, y3, dis, b3r)
    return out


# speedup vs baseline: 1.0697x; 1.0697x over previous
import functools

import jax
import jax.numpy as jnp
from jax import lax
from jax.experimental import pallas as pl
from jax.experimental.pallas import tpu as pltpu
from jax.experimental.pallas import tpu_sc as plsc

N = 10000
E = 320000
D_IN = 128
D_H = 128
D_H2 = 64
DDEG = 8

NC = 2
NS = 16
NW = NC * NS
CBP = 128
CHP = 80
EPW = CHP * CBP
EPAD = NW * EPW - E
NPAD = 10240
RPT = NPAD // NS

_SC_MESH = dict(core_axis_name="c", subcore_axis_name="s",
                num_cores=NC, num_subcores=NS)
_SC_PARAMS = pltpu.CompilerParams(use_tc_tiling_on_sc=False)


def _make_agg(D):
    @functools.partial(
        pl.kernel,
        out_type=jax.ShapeDtypeStruct((NC, NPAD, D), jnp.float32),
        mesh=plsc.VectorSubcoreMesh(**_SC_MESH),
        scratch_types=[
            pltpu.VMEM((CBP,), jnp.int32),
            pltpu.VMEM((CBP,), jnp.int32),
            pltpu.VMEM((CBP,), jnp.int32),
            pltpu.VMEM((CBP,), jnp.int32),
            pltpu.VMEM((CBP, D), jnp.float32),
            pltpu.VMEM((CBP, D), jnp.float32),
            pltpu.VMEM_SHARED((NPAD, D), jnp.float32),
            pltpu.SemaphoreType.DMA,
            pltpu.SemaphoreType.DMA,
            pltpu.SemaphoreType.DMA,
            pltpu.SemaphoreType.DMA,
        ],
        compiler_params=_SC_PARAMS,
    )
    def agg(y_hbm, src_hbm, dst_hbm, zero_hbm, out_hbm, sa, da, sb, db,
            ra, rb, acc_sh, gsa, ssa, gsb, ssb):
        c = lax.axis_index("c")
        s = lax.axis_index("s")
        w = c * NS + s
        r0 = s * RPT
        pltpu.sync_copy(zero_hbm, acc_sh.at[pl.ds(r0, RPT)])
        plsc.subcore_barrier()

        def gather(sidx, rows, gsem, k):
            pltpu.sync_copy(src_hbm.at[w, k], sidx)
            pltpu.async_copy(y_hbm.at[sidx], rows, gsem)

        gather(sa, ra, gsa, 0)
        pltpu.make_async_copy(y_hbm.at[sa], ra, gsa).wait()
        pltpu.sync_copy(dst_hbm.at[w, 0], da)
        pltpu.async_copy(ra, acc_sh.at[da], ssa, add=True)
        gather(sb, rb, gsb, 1)

        def body(t, carry):
            k0 = 2 * t
            pltpu.make_async_copy(y_hbm.at[sb], rb, gsb).wait()
            pltpu.sync_copy(dst_hbm.at[w, k0 - 1], db)
            pltpu.async_copy(rb, acc_sh.at[db], ssb, add=True)
            pltpu.make_async_copy(ra, acc_sh.at[da], ssa).wait()
            gather(sa, ra, gsa, k0)
            pltpu.make_async_copy(y_hbm.at[sa], ra, gsa).wait()
            pltpu.sync_copy(dst_hbm.at[w, k0], da)
            pltpu.async_copy(ra, acc_sh.at[da], ssa, add=True)
            pltpu.make_async_copy(rb, acc_sh.at[db], ssb).wait()
            gather(sb, rb, gsb, k0 + 1)
            return carry

        lax.fori_loop(1, CHP // 2, body, 0)
        pltpu.make_async_copy(y_hbm.at[sb], rb, gsb).wait()
        pltpu.sync_copy(dst_hbm.at[w, CHP - 1], db)
        pltpu.async_copy(rb, acc_sh.at[db], ssb, add=True)
        pltpu.make_async_copy(ra, acc_sh.at[da], ssa).wait()
        pltpu.make_async_copy(rb, acc_sh.at[db], ssb).wait()
        plsc.subcore_barrier()
        pltpu.sync_copy(acc_sh.at[pl.ds(r0, RPT)],
                        out_hbm.at[c].at[pl.ds(r0, RPT)])

    return agg


_agg128 = _make_agg(128)
_agg64 = _make_agg(64)
_agg8 = _make_agg(8)


@functools.partial(
    pl.kernel,
    out_type=jax.ShapeDtypeStruct((NC, NPAD, DDEG), jnp.float32),
    mesh=plsc.VectorSubcoreMesh(**_SC_MESH),
    scratch_types=[
        pltpu.VMEM((CHP, CBP), jnp.int32),
        pltpu.VMEM((CBP, DDEG), jnp.float32),
        pltpu.VMEM_SHARED((NPAD, DDEG), jnp.float32),
        pltpu.SemaphoreType.DMA,
    ],
    compiler_params=_SC_PARAMS,
)
def _deg_sc(ones_hbm, dst_hbm, zero_hbm, out_hbm, dst_st, rows_v, acc_sh, sem):
    c = lax.axis_index("c")
    s = lax.axis_index("s")
    w = c * NS + s
    r0 = s * RPT
    pltpu.sync_copy(dst_hbm.at[w], dst_st)
    pltpu.sync_copy(zero_hbm, acc_sh.at[pl.ds(r0, RPT)])
    pltpu.sync_copy(ones_hbm, rows_v)
    plsc.subcore_barrier()

    def body(j, carry):
        pltpu.async_copy(rows_v, acc_sh.at[dst_st.at[j]], sem, add=True)
        return carry

    lax.fori_loop(0, CHP, body, 0)

    def drain(j, carry):
        pltpu.make_async_copy(rows_v, acc_sh.at[dst_st.at[0]], sem).wait()
        return carry

    lax.fori_loop(0, CHP, drain, 0)
    plsc.subcore_barrier()
    pltpu.sync_copy(acc_sh.at[pl.ds(r0, RPT)],
                    out_hbm.at[c].at[pl.ds(r0, RPT)])


_RB = 1000
_GRID = N // _RB


def _part_spec(core, d):
    return pl.BlockSpec((1, _RB, d), lambda i, c=core: (c, i, 0))


def _row_spec(d):
    return pl.BlockSpec((_RB, d), lambda i: (i, 0))


def _full_spec(shape):
    return pl.BlockSpec(shape, lambda i: tuple(0 for _ in shape))


def _bf16_dot(a, b):
    return jnp.dot(a.astype(jnp.bfloat16), b.astype(jnp.bfloat16),
                   preferred_element_type=jnp.float32)


def _k0_body(x, w1, xw_ref):
    xw_ref[...] = _bf16_dot(x[...], w1[...])


def _k1_body(d0, d1, xw, y_ref, dis_ref):
    deg = d0[0][:, 0:1] + d1[0][:, 0:1] + 1.0
    dis = 1.0 / jnp.sqrt(deg)
    dis_ref[...] = dis
    y_ref[...] = dis * xw[...]


def _k2_body(p0, p1, y1, dis, b1, w2, y2_ref):
    h = jnp.maximum(dis[...] * (p0[0] + p1[0] + y1[...]) + b1[...], 0.0)
    y2_ref[...] = dis[...] * _bf16_dot(h, w2[...])


def _k3_body(q0, q1, y2, dis, b2, w3p, y3_ref):
    h = jnp.maximum(dis[...] * (q0[0] + q1[0] + y2[...]) + b2[...], 0.0)
    y3_ref[...] = dis[...] * _bf16_dot(h, w3p[...])


def _k4_body(r0, r1, y3, dis, b3, out_ref):
    t = r0[0][:, 0:1] + r1[0][:, 0:1] + y3[:, 0:1]
    out_ref[...] = dis[...] * t + b3[...]


def kernel(x, edge_index, W1, b1, W2, b2, W3, b3):
    f32 = jnp.float32
    i32 = jnp.int32
    src = jnp.concatenate(
        [edge_index[0], jnp.zeros((EPAD,), i32)]).reshape(NW, CHP, CBP)
    dst = jnp.concatenate(
        [edge_index[1], jnp.full((EPAD,), N, i32)]).reshape(NW, CHP, CBP)
    onesb = jnp.ones((CBP, DDEG), f32)
    zdeg = jnp.zeros((RPT, DDEG), f32)
    z64 = jnp.zeros((RPT, 64), f32)
    z128 = jnp.zeros((RPT, 128), f32)
    b1r = b1.reshape(1, D_H)
    b2r = b2.reshape(1, D_H2)
    b3r = b3.reshape(1, 1)

    degp = _deg_sc(onesb, dst, zdeg)
    xw1 = pl.pallas_call(
        _k0_body,
        grid=(_GRID,),
        in_specs=[_row_spec(D_IN), _full_spec((D_IN, D_H))],
        out_specs=_row_spec(D_H),
        out_shape=jax.ShapeDtypeStruct((N, D_H), f32),
    )(x, W1)
    y1, dis = pl.pallas_call(
        _k1_body,
        grid=(_GRID,),
        in_specs=[_part_spec(0, DDEG), _part_spec(1, DDEG), _row_spec(D_H)],
        out_specs=[_row_spec(D_H), _row_spec(1)],
        out_shape=[jax.ShapeDtypeStruct((N, D_H), f32),
                   jax.ShapeDtypeStruct((N, 1), f32)],
    )(degp, degp, xw1)

    p = _agg128(y1, src, dst, z128)
    y2 = pl.pallas_call(
        _k2_body,
        grid=(_GRID,),
        in_specs=[_part_spec(0, D_H), _part_spec(1, D_H), _row_spec(D_H),
                  _row_spec(1), _full_spec((1, D_H)), _full_spec((D_H, D_H2))],
        out_specs=_row_spec(D_H2),
        out_shape=jax.ShapeDtypeStruct((N, D_H2), f32),
    )(p, p, y1, dis, b1r, W2)

    q = _agg64(y2, src, dst, z64)
    w3p = jnp.pad(W3, ((0, 0), (0, DDEG - 1)))
    y3 = pl.pallas_call(
        _k3_body,
        grid=(_GRID,),
        in_specs=[_part_spec(0, D_H2), _part_spec(1, D_H2), _row_spec(D_H2),
                  _row_spec(1), _full_spec((1, D_H2)),
                  _full_spec((D_H2, DDEG))],
        out_specs=_row_spec(DDEG),
        out_shape=jax.ShapeDtypeStruct((N, DDEG), f32),
    )(q, q, y2, dis, b2r, w3p)

    r = _agg8(y3, src, dst, zdeg)
    out = pl.pallas_call(
        _k4_body,
        grid=(_GRID,),
        in_specs=[_part_spec(0, DDEG), _part_spec(1, DDEG), _row_spec(DDEG),
                  _row_spec(1), _full_spec((1, 1))],
        out_specs=_row_spec(1),
        out_shape=jax.ShapeDtypeStruct((N, 1), f32),
    )(r, r, y3, dis, b3r)
    return out
